# baseline (speedup 1.0000x reference)
import functools

import jax
import jax.numpy as jnp
from jax import lax
from jax.experimental import pallas as pl
from jax.experimental.pallas import tpu as pltpu
from jax.experimental.pallas import tpu_sc as plsc

V = 1000000
D = 64
B = 16384
NNEG = 20

NC = 2
NS = 16
NW = NC * NS
NPW = B // NW
E = 32
NCHUNK = NPW // E
NEG_PER_CHUNK = E * NNEG
NEG_IDX_ROWS = NEG_PER_CHUNK // 128


def _skipgram_body(in_table, out_table, in_idx, pos_idx, neg_idx,
                   pos_out, neg_out,
                   in_idx_v, pos_idx_v, neg_idx_v,
                   ri0, ri1, rp0, rp1, rn0, rn1,
                   po0, po1, no0, no1, sem0, sem1):
    wid = lax.axis_index("s") * NC + lax.axis_index("c")
    base = wid * NPW

    pltpu.sync_copy(in_idx.at[pl.ds(base, NPW)], in_idx_v)
    pltpu.sync_copy(pos_idx.at[pl.ds(base, NPW)], pos_idx_v)
    pltpu.sync_copy(neg_idx.at[pl.ds(wid * (NPW * NNEG // 128), NPW * NNEG // 128)],
                    neg_idx_v)

    bufs = ((ri0, rp0, rn0, po0, no0, sem0),
            (ri1, rp1, rn1, po1, no1, sem1))

    def fire(k, slot):
        ri, rp, rn, _, _, sem = bufs[slot]
        o = k * E
        pltpu.async_copy(in_table.at[in_idx_v.at[pl.ds(o, E)]], ri, sem)
        pltpu.async_copy(out_table.at[pos_idx_v.at[pl.ds(o, E)]], rp, sem)
        for i in range(NEG_IDX_ROWS):
            pltpu.async_copy(out_table.at[neg_idx_v.at[k * NEG_IDX_ROWS + i]],
                             rn.at[pl.ds(i * 128, 128)], sem)

    def drain(slot):
        ri, rp, rn, _, _, sem = bufs[slot]
        pltpu.make_async_copy(in_table.at[pl.ds(0, E)], ri, sem).wait()
        pltpu.make_async_copy(out_table.at[pl.ds(0, E)], rp, sem).wait()
        pltpu.make_async_copy(out_table.at[pl.ds(0, NEG_PER_CHUNK)], rn, sem).wait()

    def compute(k, slot):
        ri, rp, rn, po, no, _ = bufs[slot]
        lanes = lax.iota(jnp.int32, 16)
        for g in range(E // 16):
            erow = lanes + (16 * g)
            acc_pos = jnp.zeros((16,), jnp.float32)
            for dc in range(D // 16):
                d0 = dc * 16
                in_t = [plsc.load_gather(ri, [erow, jnp.full((16,), d0 + t, jnp.int32)])
                        for t in range(16)]
                for t in range(16):
                    pcol = plsc.load_gather(rp, [erow, jnp.full((16,), d0 + t, jnp.int32)])
                    acc_pos = acc_pos + in_t[t] * pcol

                def jbody(j, _, dc=dc, in_t=in_t, erow=erow, rn=rn, no=no):
                    r = erow * NNEG + j
                    acc = jnp.zeros((16,), jnp.float32)
                    for t in range(16):
                        ncol = plsc.load_gather(
                            rn, [r, jnp.full((16,), dc * 16 + t, jnp.int32)])
                        acc = acc + in_t[t] * ncol
                    if dc == 0:
                        plsc.store_scatter(no, [r], acc)
                    else:
                        plsc.addupdate_scatter(no, [r], acc)
                    return 0

                lax.fori_loop(0, NNEG, jbody, 0, unroll=False)
            po[pl.ds(16 * g, 16)] = acc_pos
        pltpu.sync_copy(po, pos_out.at[pl.ds(base + k * E, E)])
        pltpu.sync_copy(no, neg_out.at[pl.ds((base + k * E) * NNEG, NEG_PER_CHUNK)])

    fire(0, 0)

    def chunk_step(c, _):
        for b in range(2):
            k = c + b
            drain(b)
            pl.when(k + 1 < NCHUNK)(lambda: fire(k + 1, (b + 1) % 2))
            compute(k, b)
        return 0

    lax.fori_loop(0, NCHUNK // 2, lambda c, carry: chunk_step(2 * c, carry), 0,
                  unroll=False)


@functools.partial(jax.jit, static_argnames=())
def _skipgram(in_table, out_table, in_idx, pos_idx, neg_idx):
    f = pl.kernel(
        _skipgram_body,
        out_type=[jax.ShapeDtypeStruct((B,), jnp.float32),
                  jax.ShapeDtypeStruct((B * NNEG,), jnp.float32)],
        mesh=plsc.VectorSubcoreMesh(core_axis_name="c", subcore_axis_name="s"),
        compiler_params=pltpu.CompilerParams(needs_layout_passes=False,
                                             use_tc_tiling_on_sc=False),
        scratch_types=[
            pltpu.VMEM((NPW,), jnp.int32),
            pltpu.VMEM((NPW,), jnp.int32),
            pltpu.VMEM((NPW * NNEG // 128, 128), jnp.int32),
            pltpu.VMEM((E, D), jnp.float32),
            pltpu.VMEM((E, D), jnp.float32),
            pltpu.VMEM((E, D), jnp.float32),
            pltpu.VMEM((E, D), jnp.float32),
            pltpu.VMEM((NEG_PER_CHUNK, D), jnp.float32),
            pltpu.VMEM((NEG_PER_CHUNK, D), jnp.float32),
            pltpu.VMEM((E,), jnp.float32),
            pltpu.VMEM((E,), jnp.float32),
            pltpu.VMEM((NEG_PER_CHUNK,), jnp.float32),
            pltpu.VMEM((NEG_PER_CHUNK,), jnp.float32),
            pltpu.SemaphoreType.DMA,
            pltpu.SemaphoreType.DMA,
        ],
    )
    return f(in_table, out_table, in_idx, pos_idx, neg_idx)


def kernel(in_table, out_table, inputs, contexts, negatives):
    in_idx = inputs.reshape(B)
    pos_idx = contexts.reshape(B)
    neg_idx = negatives.reshape(B * NNEG // 128, 128)
    pos, neg = _skipgram(in_table, out_table, in_idx, pos_idx, neg_idx)
    return pos, neg.reshape(B, NNEG)
